# Initial kernel scaffold; baseline (speedup 1.0000x reference)
import jax
import jax.numpy as jnp
from jax.experimental import pallas as pl


def kernel(x, edge_index, edge_attr, W, batch_ids, emb_table, pe_in_w, pe_in_b, pe_self_w1, pe_nbr_w1, pe_b1, pe_self_w2, pe_nbr_w2, pe_b2, rho_w1, rho_b1, rho_w2, rho_b2, pe_emb_w, pe_emb_b, edge_w1, edge_b1, eps1, mlp1_w1, mlp1_b1, mlp1_w2, mlp1_b2, edge_w2, edge_b2, eps2, mlp2_w1, mlp2_b1, mlp2_w2, mlp2_b2, ro_w, ro_b):
    raise NotImplementedError("write your pallas kernel here")


# speedup vs baseline: 29.5316x; 29.5316x over previous
import functools

import jax
import jax.numpy as jnp
from jax import lax
from jax.experimental import pallas as pl
from jax.experimental.pallas import tpu as pltpu
from jax.experimental.pallas import tpu_sc as plsc

N = 10000
E = 320000
M = 8
PH = 8
D = 128
DE = 16
PE_OUT = 64
G = 128
OUT = 64
NT = 64
MPH = M * PH
HD = D // 2

NCORES = 2
NSUB = 16
NTILES = NCORES * NSUB
EPT = E // NTILES
CHUNK = 80
NCH = EPT // CHUNK
STRIPE = 624
TAIL = N - NSUB * STRIPE
TAIL_OFF = NSUB * STRIPE


@functools.lru_cache(maxsize=None)
def _make_segsum(F, with_e):
    mesh = plsc.VectorSubcoreMesh(core_axis_name="c", subcore_axis_name="s",
                                  num_cores=NCORES, num_subcores=NSUB)

    def body(*refs):
        if with_e:
            (feat_hbm, src_hbm, dst_hbm, e_hbm, out_hbm,
             sidx, didx, rows, ebuf, bounce, agg, sem) = refs
        else:
            (feat_hbm, src_hbm, dst_hbm, out_hbm,
             sidx, didx, rows, bounce, agg, sem) = refs
        c = lax.axis_index("c")
        s = lax.axis_index("s")
        wid = c * NSUB + s

        def zrow(i, _):
            for j in range(F // 16):
                bounce[i, pl.ds(j * 16, 16)] = jnp.zeros((16,), jnp.float32)
            return 0
        lax.fori_loop(0, STRIPE, zrow, 0)
        pltpu.sync_copy(bounce, agg.at[pl.ds(s * STRIPE, STRIPE)])

        @pl.when(s == 0)
        def _():
            pltpu.sync_copy(bounce.at[pl.ds(0, TAIL)],
                            agg.at[pl.ds(TAIL_OFF, TAIL)])
        plsc.subcore_barrier()

        def chunk(i, _):
            base = wid * EPT + i * CHUNK
            pltpu.sync_copy(src_hbm.at[pl.ds(base, CHUNK)], sidx)
            pltpu.sync_copy(dst_hbm.at[pl.ds(base, CHUNK)], didx)
            pltpu.async_copy(feat_hbm.at[sidx], rows, sem).wait()
            if with_e:
                pltpu.sync_copy(e_hbm.at[pl.ds(base, CHUNK)], ebuf)

                def row(r, _):
                    for j in range(F // 16):
                        sl = pl.ds(j * 16, 16)
                        rows[r, sl] = jnp.maximum(rows[r, sl] + ebuf[r, sl], 0.0)
                    return 0
                lax.fori_loop(0, CHUNK, row, 0)
            pltpu.sync_copy(rows, agg.at[didx], add=True)
            return 0
        lax.fori_loop(0, NCH, chunk, 0)

        plsc.subcore_barrier()
        pltpu.sync_copy(agg.at[pl.ds(s * STRIPE, STRIPE)], bounce)
        pltpu.sync_copy(bounce, out_hbm.at[c, pl.ds(s * STRIPE, STRIPE)])

        @pl.when(s == 0)
        def _():
            pltpu.sync_copy(agg.at[pl.ds(TAIL_OFF, TAIL)],
                            bounce.at[pl.ds(0, TAIL)])
            pltpu.sync_copy(bounce.at[pl.ds(0, TAIL)],
                            out_hbm.at[c, pl.ds(TAIL_OFF, TAIL)])

    scratch = [
        pltpu.VMEM((CHUNK,), jnp.int32),
        pltpu.VMEM((CHUNK,), jnp.int32),
        pltpu.VMEM((CHUNK, F), jnp.float32),
    ]
    if with_e:
        scratch.append(pltpu.VMEM((CHUNK, F), jnp.float32))
    scratch += [
        pltpu.VMEM((STRIPE, F), jnp.float32),
        pltpu.VMEM_SHARED((N, F), jnp.float32),
        pltpu.SemaphoreType.DMA,
    ]

    return pl.kernel(
        body,
        out_type=jax.ShapeDtypeStruct((NCORES, N, F), jnp.float32),
        mesh=mesh,
        scratch_types=scratch,
        compiler_params=pltpu.CompilerParams(use_tc_tiling_on_sc=False),
    )


def _dot(a, b):
    return jnp.dot(a, b, preferred_element_type=jnp.float32)


def _tc_prep(x_ref, w_ref, emb_ref, a0_ref, b0_ref, xn_ref, hp0_ref):
    onehot = (x_ref[...] == lax.broadcasted_iota(jnp.int32, (1, NT), 1))
    xn_ref[...] = _dot(onehot.astype(jnp.float32), emb_ref[...])
    hp0_ref[...] = jnp.maximum(_dot(w_ref[...], a0_ref[...]) + b0_ref[...], 0.0)


def _tc_pe_mid(hp_ref, pa_ref, pb_ref, ks_ref, kn_ref, b_ref, out_ref):
    agg = pa_ref[...] + pb_ref[...]
    out_ref[...] = jnp.maximum(
        _dot(hp_ref[...], ks_ref[...]) + _dot(agg, kn_ref[...]) + b_ref[...], 0.0)


def _tc_pe_final(hp_ref, pa_ref, pb_ref, ks_ref, kn_ref, b_ref, p_ref,
                 r1_ref, rb1_ref, r2_ref, rb2_ref, xn_ref, pw_ref, pb2_ref,
                 outa_ref, outb_ref):
    agg = pa_ref[...] + pb_ref[...]
    hp2 = jnp.maximum(
        _dot(hp_ref[...], ks_ref[...]) + _dot(agg, kn_ref[...]) + b_ref[...], 0.0)
    pooled = _dot(hp2, p_ref[...])
    pe = _dot(jnp.maximum(_dot(pooled, r1_ref[...]) + rb1_ref[...], 0.0),
              r2_ref[...]) + rb2_ref[...]
    x1 = xn_ref[...] + _dot(pe, pw_ref[...]) + pb2_ref[...]
    outa_ref[...] = x1[:, :HD]
    outb_ref[...] = x1[:, HD:]


def _tc_edges(ea_ref, w1_ref, b1_ref, w2_ref, b2_ref,
              e1a_ref, e1b_ref, e2a_ref, e2b_ref):
    ea = ea_ref[...]
    e1 = _dot(ea, w1_ref[...]) + b1_ref[...]
    e2 = _dot(ea, w2_ref[...]) + b2_ref[...]
    e1a_ref[...] = e1[:, :HD]
    e1b_ref[...] = e1[:, HD:]
    e2a_ref[...] = e2[:, :HD]
    e2b_ref[...] = e2[:, HD:]


def _tc_mlp(xa_ref, xb_ref, paa_ref, pab_ref, pba_ref, pbb_ref, eps_ref,
            w1_ref, b1_ref, w2_ref, b2_ref, outa_ref, outb_ref):
    sc = 1.0 + eps_ref[0, 0]
    ha = sc * xa_ref[...] + paa_ref[...] + pab_ref[...]
    hb = sc * xb_ref[...] + pba_ref[...] + pbb_ref[...]
    w1 = w1_ref[...]
    t = jnp.maximum(_dot(ha, w1[:HD, :]) + _dot(hb, w1[HD:, :]) + b1_ref[...],
                    0.0)
    x2 = _dot(t, w2_ref[...]) + b2_ref[...]
    outa_ref[...] = x2[:, :HD]
    outb_ref[...] = x2[:, HD:]


def _tc_final(xa_ref, xb_ref, paa_ref, pab_ref, pba_ref, pbb_ref, eps_ref,
              w1_ref, b1_ref, w2_ref, b2_ref,
              bid_ref, row_ref, rob_ref, out_ref):
    sc = 1.0 + eps_ref[0, 0]
    ha = sc * xa_ref[...] + paa_ref[...] + pab_ref[...]
    hb = sc * xb_ref[...] + pba_ref[...] + pbb_ref[...]
    w1 = w1_ref[...]
    t = jnp.maximum(_dot(ha, w1[:HD, :]) + _dot(hb, w1[HD:, :]) + b1_ref[...],
                    0.0)
    x3 = _dot(t, w2_ref[...]) + b2_ref[...]
    onehot = (bid_ref[...] == lax.broadcasted_iota(jnp.int32, (1, G), 1))
    onehot = onehot.astype(jnp.float32)
    sums = lax.dot_general(onehot, x3, (((0,), (0,)), ((), ())),
                           preferred_element_type=jnp.float32)
    counts = lax.dot_general(onehot, jnp.ones((N, 1), jnp.float32),
                             (((0,), (0,)), ((), ())),
                             preferred_element_type=jnp.float32)
    g = sums / jnp.clip(counts, 1.0, None)
    out_ref[...] = _dot(g, row_ref[...]) + rob_ref[...]


def _call(body, out_shapes, *args):
    return pl.pallas_call(
        body,
        out_shape=out_shapes,
    )(*args)


def kernel(x, edge_index, edge_attr, W, batch_ids, emb_table, pe_in_w,
           pe_in_b, pe_self_w1, pe_nbr_w1, pe_b1, pe_self_w2, pe_nbr_w2,
           pe_b2, rho_w1, rho_b1, rho_w2, rho_b2, pe_emb_w, pe_emb_b,
           edge_w1, edge_b1, eps1, mlp1_w1, mlp1_b1, mlp1_w2, mlp1_b2,
           edge_w2, edge_b2, eps2, mlp2_w1, mlp2_b1, mlp2_w2, mlp2_b2,
           ro_w, ro_b):
    src = edge_index[0]
    dst = edge_index[1]
    eye = jnp.eye(PH, dtype=jnp.float32)

    a0 = jnp.kron(eye, pe_in_w)
    b0 = jnp.tile(pe_in_b, M)[None, :]
    ks1 = jnp.kron(eye, pe_self_w1)
    kn1 = jnp.kron(eye, pe_nbr_w1)
    b1 = jnp.tile(pe_b1, M)[None, :]
    ks2 = jnp.kron(eye, pe_self_w2)
    kn2 = jnp.kron(eye, pe_nbr_w2)
    b2 = jnp.tile(pe_b2, M)[None, :]
    psum = jnp.tile(eye, (M, 1))

    xn, hp0 = _call(
        _tc_prep,
        (jax.ShapeDtypeStruct((N, D), jnp.float32),
         jax.ShapeDtypeStruct((N, MPH), jnp.float32)),
        x, W, emb_table, a0, b0)

    p = _make_segsum(MPH, False)(hp0, src, dst)
    hp1 = _call(_tc_pe_mid, jax.ShapeDtypeStruct((N, MPH), jnp.float32),
                hp0, p[0], p[1], ks1, kn1, b1)

    p = _make_segsum(MPH, False)(hp1, src, dst)
    half = jax.ShapeDtypeStruct((N, HD), jnp.float32)
    x1a, x1b = _call(_tc_pe_final, (half, half),
                     hp1, p[0], p[1], ks2, kn2, b2, psum,
                     rho_w1, rho_b1[None, :], rho_w2, rho_b2[None, :],
                     xn, pe_emb_w, pe_emb_b[None, :])

    EB = 8000
    ehalf = pl.BlockSpec((EB, HD), lambda i: (i, 0))
    e1a, e1b, e2a, e2b = pl.pallas_call(
        _tc_edges,
        grid=(E // EB,),
        in_specs=[
            pl.BlockSpec((EB, DE), lambda i: (i, 0)),
            pl.BlockSpec((DE, D), lambda i: (0, 0)),
            pl.BlockSpec((1, D), lambda i: (0, 0)),
            pl.BlockSpec((DE, D), lambda i: (0, 0)),
            pl.BlockSpec((1, D), lambda i: (0, 0)),
        ],
        out_specs=[ehalf, ehalf, ehalf, ehalf],
        out_shape=[jax.ShapeDtypeStruct((E, HD), jnp.float32)] * 4,
    )(edge_attr, edge_w1, edge_b1[None, :], edge_w2, edge_b2[None, :])

    pa = _make_segsum(HD, True)(x1a, src, dst, e1a)
    pb = _make_segsum(HD, True)(x1b, src, dst, e1b)
    x2a, x2b = _call(_tc_mlp, (half, half),
                     x1a, x1b, pa[0], pa[1], pb[0], pb[1],
                     jnp.reshape(eps1, (1, 1)),
                     mlp1_w1, mlp1_b1[None, :], mlp1_w2, mlp1_b2[None, :])

    pa = _make_segsum(HD, True)(x2a, src, dst, e2a)
    pb = _make_segsum(HD, True)(x2b, src, dst, e2b)
    out = _call(_tc_final, jax.ShapeDtypeStruct((G, OUT), jnp.float32),
                x2a, x2b, pa[0], pa[1], pb[0], pb[1],
                jnp.reshape(eps2, (1, 1)),
                mlp2_w1, mlp2_b1[None, :], mlp2_w2, mlp2_b2[None, :],
                batch_ids[:, None], ro_w, ro_b[None, :])
    return out
